# Initial kernel scaffold; baseline (speedup 1.0000x reference)
import jax
import jax.numpy as jnp
from jax.experimental import pallas as pl


def kernel(loc_data, conf_data, dboxes, target_bboxes, target_labels):
    raise NotImplementedError("write your pallas kernel here")


# speedup vs baseline: 10.0072x; 10.0072x over previous
import jax
import jax.numpy as jnp
from jax.experimental import pallas as pl
from jax.experimental.pallas import tpu as pltpu


def _row_kernel(tb_ref, tl_ref, conf_ref, locT_ref, dbT_ref,
                ce_neg_ref, nump_ref, posce_ref, lossl_ref):
    conf = conf_ref[0]
    locT = locT_ref[0]
    db0 = dbT_ref[0, :]
    db1 = dbT_ref[1, :]
    db2 = dbT_ref[2, :]
    db3 = dbT_ref[3, :]
    area_d = (db0 - db2) * (db1 - db3)

    num_t = tb_ref.shape[1]
    best_iou = None
    for o in range(num_t):
        tx0 = tb_ref[0, o, 0]
        tx1 = tb_ref[0, o, 1]
        tx2 = tb_ref[0, o, 2]
        tx3 = tb_ref[0, o, 3]
        lab = tl_ref[0, 0, o]
        w = jnp.clip(jnp.minimum(tx2, db2) - jnp.maximum(tx0, db0), 0.0, None)
        h = jnp.clip(jnp.minimum(tx3, db3) - jnp.maximum(tx1, db1), 0.0, None)
        inter = w * h
        area_t = (tx0 - tx2) * (tx1 - tx3)
        iou = inter / (area_t + area_d - inter)
        if o == 0:
            best_iou = iou
            bx0 = jnp.full_like(db0, tx0)
            bx1 = jnp.full_like(db0, tx1)
            bx2 = jnp.full_like(db0, tx2)
            bx3 = jnp.full_like(db0, tx3)
            blab = jnp.full(db0.shape, lab, dtype=jnp.int32)
        else:
            upd = iou > best_iou
            best_iou = jnp.where(upd, iou, best_iou)
            bx0 = jnp.where(upd, tx0, bx0)
            bx1 = jnp.where(upd, tx1, bx1)
            bx2 = jnp.where(upd, tx2, bx2)
            bx3 = jnp.where(upd, tx3, bx3)
            blab = jnp.where(upd, lab, blab)

    c = jnp.where(best_iou < 0.5, 0, blab + 1)
    pos = c > 0

    dsx = db2 - db0
    dsy = db3 - db1
    l0 = jnp.trunc((bx0 - db0 + bx2 - db2) * (0.5 * 10.0) / dsx)
    l1 = jnp.trunc((bx1 - db1 + bx3 - db3) * (0.5 * 10.0) / dsy)
    l2 = jnp.trunc(jnp.log((bx2 - bx0) / dsx) * 5.0)
    l3 = jnp.trunc(jnp.log((bx3 - bx1) / dsy) * 5.0)

    loss_l = jnp.float32(0.0)
    for j, lj in enumerate((l0, l1, l2, l3)):
        d = jnp.abs(locT[j] - lj)
        sl1 = jnp.where(d < 1.0, 0.5 * d * d, d - 0.5)
        loss_l = loss_l + jnp.sum(jnp.where(pos, sl1, 0.0))

    m = jnp.max(conf, axis=1)
    s = jnp.sum(jnp.exp(conf - m[:, None]), axis=1)
    lane = jax.lax.broadcasted_iota(jnp.int32, conf.shape, 1)
    picked = jnp.sum(jnp.where(lane == c[:, None], conf, 0.0), axis=1)
    ce = m + jnp.log(s) - picked

    ce_neg = jnp.where(pos, 0.0, ce)
    ce_neg = jnp.maximum(ce_neg, 0.0)
    ce_neg_ref[0, 0, :] = ce_neg
    nump_ref[0, 0, 0] = jnp.sum(pos.astype(jnp.int32))
    posce_ref[0, 0, 0] = jnp.sum(jnp.where(pos, ce, 0.0))
    lossl_ref[0, 0, 0] = loss_l


def _final_kernel(ce_neg_ref, nump_ref, posce_ref, lossl_ref,
                  out_l_ref, out_c_ref):
    cn = ce_neg_ref[...][:, 0, :]
    num_dbox = cn.shape[1]
    bits = jax.lax.bitcast_convert_type(cn, jnp.int32)
    np_vec = nump_ref[...][:, 0, :]
    k = jnp.minimum(np_vec * 3, num_dbox)

    t = jnp.zeros_like(np_vec)
    for bit in range(30, -1, -1):
        cand = t | (1 << bit)
        cnt = jnp.sum((bits >= cand).astype(jnp.int32), axis=1, keepdims=True)
        t = jnp.where(cnt >= k, cand, t)

    thr = jax.lax.bitcast_convert_type(t, jnp.float32)
    gt = bits > t
    cnt_gt = jnp.sum(gt.astype(jnp.int32), axis=1, keepdims=True)
    sum_gt = jnp.sum(jnp.where(gt, cn, 0.0), axis=1, keepdims=True)
    topk = sum_gt + (k - cnt_gt).astype(jnp.float32) * thr
    topk = jnp.where(k > 0, topk, 0.0)

    n_total = jnp.sum(np_vec).astype(jnp.float32)
    out_l_ref[0, 0] = jnp.sum(lossl_ref[...]) / n_total
    out_c_ref[0, 0] = (jnp.sum(posce_ref[...]) + jnp.sum(topk)) / n_total


def kernel(loc_data, conf_data, dboxes, target_bboxes, target_labels):
    b, d, c = conf_data.shape
    locT = jnp.swapaxes(loc_data, 1, 2)
    dbT = dboxes.T
    tl = target_labels.astype(jnp.int32)[:, None, :]
    o = tl.shape[2]

    ce_neg, nump, posce, lossl = pl.pallas_call(
        _row_kernel,
        grid=(b,),
        in_specs=[
            pl.BlockSpec((1, o, 4), lambda i: (i, 0, 0),
                         memory_space=pltpu.SMEM),
            pl.BlockSpec((1, 1, o), lambda i: (i, 0, 0),
                         memory_space=pltpu.SMEM),
            pl.BlockSpec((1, d, c), lambda i: (i, 0, 0)),
            pl.BlockSpec((1, 4, d), lambda i: (i, 0, 0)),
            pl.BlockSpec((4, d), lambda i: (0, 0)),
        ],
        out_specs=[
            pl.BlockSpec((1, 1, d), lambda i: (i, 0, 0)),
            pl.BlockSpec((1, 1, 1), lambda i: (i, 0, 0),
                         memory_space=pltpu.SMEM),
            pl.BlockSpec((1, 1, 1), lambda i: (i, 0, 0),
                         memory_space=pltpu.SMEM),
            pl.BlockSpec((1, 1, 1), lambda i: (i, 0, 0),
                         memory_space=pltpu.SMEM),
        ],
        out_shape=[
            jax.ShapeDtypeStruct((b, 1, d), jnp.float32),
            jax.ShapeDtypeStruct((b, 1, 1), jnp.int32),
            jax.ShapeDtypeStruct((b, 1, 1), jnp.float32),
            jax.ShapeDtypeStruct((b, 1, 1), jnp.float32),
        ],
    )(target_bboxes, tl, conf_data, locT, dbT)

    out_l, out_c = pl.pallas_call(
        _final_kernel,
        out_specs=[
            pl.BlockSpec(memory_space=pltpu.SMEM),
            pl.BlockSpec(memory_space=pltpu.SMEM),
        ],
        out_shape=[
            jax.ShapeDtypeStruct((1, 1), jnp.float32),
            jax.ShapeDtypeStruct((1, 1), jnp.float32),
        ],
    )(ce_neg, nump, posce, lossl)

    return (out_l[0, 0], out_c[0, 0])


# speedup vs baseline: 19.8271x; 1.9813x over previous
import jax
import jax.numpy as jnp
from jax.experimental import pallas as pl
from jax.experimental.pallas import tpu as pltpu


def _row_kernel(tb_ref, tl_ref, conf_ref, locT_ref, dbT_ref,
                ce_neg_ref, nump_ref, posce_ref, lossl_ref):
    conf = conf_ref[0]
    locT = locT_ref[0]
    db0 = dbT_ref[0, :]
    db1 = dbT_ref[1, :]
    db2 = dbT_ref[2, :]
    db3 = dbT_ref[3, :]
    area_d = (db0 - db2) * (db1 - db3)

    num_t = tb_ref.shape[1]
    best_iou = None
    for o in range(num_t):
        tx0 = tb_ref[0, o, 0]
        tx1 = tb_ref[0, o, 1]
        tx2 = tb_ref[0, o, 2]
        tx3 = tb_ref[0, o, 3]
        lab = tl_ref[0, 0, o]
        w = jnp.clip(jnp.minimum(tx2, db2) - jnp.maximum(tx0, db0), 0.0, None)
        h = jnp.clip(jnp.minimum(tx3, db3) - jnp.maximum(tx1, db1), 0.0, None)
        inter = w * h
        area_t = (tx0 - tx2) * (tx1 - tx3)
        iou = inter / (area_t + area_d - inter)
        if o == 0:
            best_iou = iou
            bx0 = jnp.full_like(db0, tx0)
            bx1 = jnp.full_like(db0, tx1)
            bx2 = jnp.full_like(db0, tx2)
            bx3 = jnp.full_like(db0, tx3)
            blab = jnp.full(db0.shape, lab, dtype=jnp.int32)
        else:
            upd = iou > best_iou
            best_iou = jnp.where(upd, iou, best_iou)
            bx0 = jnp.where(upd, tx0, bx0)
            bx1 = jnp.where(upd, tx1, bx1)
            bx2 = jnp.where(upd, tx2, bx2)
            bx3 = jnp.where(upd, tx3, bx3)
            blab = jnp.where(upd, lab, blab)

    c = jnp.where(best_iou < 0.5, 0, blab + 1)
    pos = c > 0

    dsx = db2 - db0
    dsy = db3 - db1
    l0 = jnp.trunc((bx0 - db0 + bx2 - db2) * (0.5 * 10.0) / dsx)
    l1 = jnp.trunc((bx1 - db1 + bx3 - db3) * (0.5 * 10.0) / dsy)
    l2 = jnp.trunc(jnp.log((bx2 - bx0) / dsx) * 5.0)
    l3 = jnp.trunc(jnp.log((bx3 - bx1) / dsy) * 5.0)

    loss_l = jnp.float32(0.0)
    for j, lj in enumerate((l0, l1, l2, l3)):
        d = jnp.abs(locT[j] - lj)
        sl1 = jnp.where(d < 1.0, 0.5 * d * d, d - 0.5)
        loss_l = loss_l + jnp.sum(jnp.where(pos, sl1, 0.0))

    m = jnp.max(conf, axis=0)
    s = jnp.sum(jnp.exp(conf - m[None, :]), axis=0)
    row = jax.lax.broadcasted_iota(jnp.int32, conf.shape, 0)
    picked = jnp.sum(jnp.where(row == c[None, :], conf, 0.0), axis=0)
    ce = m + jnp.log(s) - picked

    ce_neg = jnp.where(pos, 0.0, ce)
    ce_neg = jnp.maximum(ce_neg, 0.0)
    ce_neg_ref[0, 0, :] = ce_neg
    nump_ref[0, 0, 0] = jnp.sum(pos.astype(jnp.int32))
    posce_ref[0, 0, 0] = jnp.sum(jnp.where(pos, ce, 0.0))
    lossl_ref[0, 0, 0] = loss_l


def _final_kernel(ce_neg_ref, nump_ref, posce_ref, lossl_ref,
                  out_l_ref, out_c_ref):
    cn = ce_neg_ref[...][:, 0, :]
    num_dbox = cn.shape[1]
    bits = jax.lax.bitcast_convert_type(cn, jnp.int32)
    np_vec = nump_ref[...][:, 0, :]
    k = jnp.minimum(np_vec * 3, num_dbox)

    t = jnp.zeros_like(np_vec)
    for bit in range(30, -1, -1):
        cand = t | (1 << bit)
        cnt = jnp.sum((bits >= cand).astype(jnp.int32), axis=1, keepdims=True)
        t = jnp.where(cnt >= k, cand, t)

    thr = jax.lax.bitcast_convert_type(t, jnp.float32)
    gt = bits > t
    cnt_gt = jnp.sum(gt.astype(jnp.int32), axis=1, keepdims=True)
    sum_gt = jnp.sum(jnp.where(gt, cn, 0.0), axis=1, keepdims=True)
    topk = sum_gt + (k - cnt_gt).astype(jnp.float32) * thr
    topk = jnp.where(k > 0, topk, 0.0)

    n_total = jnp.sum(np_vec).astype(jnp.float32)
    out_l_ref[0, 0] = jnp.sum(lossl_ref[...]) / n_total
    out_c_ref[0, 0] = (jnp.sum(posce_ref[...]) + jnp.sum(topk)) / n_total


def kernel(loc_data, conf_data, dboxes, target_bboxes, target_labels):
    b, d, c = conf_data.shape
    confT = jnp.swapaxes(conf_data, 1, 2)
    locT = jnp.swapaxes(loc_data, 1, 2)
    dbT = dboxes.T
    tl = target_labels.astype(jnp.int32)[:, None, :]
    o = tl.shape[2]

    ce_neg, nump, posce, lossl = pl.pallas_call(
        _row_kernel,
        grid=(b,),
        in_specs=[
            pl.BlockSpec((1, o, 4), lambda i: (i, 0, 0),
                         memory_space=pltpu.SMEM),
            pl.BlockSpec((1, 1, o), lambda i: (i, 0, 0),
                         memory_space=pltpu.SMEM),
            pl.BlockSpec((1, c, d), lambda i: (i, 0, 0)),
            pl.BlockSpec((1, 4, d), lambda i: (i, 0, 0)),
            pl.BlockSpec((4, d), lambda i: (0, 0)),
        ],
        out_specs=[
            pl.BlockSpec((1, 1, d), lambda i: (i, 0, 0)),
            pl.BlockSpec((1, 1, 1), lambda i: (i, 0, 0),
                         memory_space=pltpu.SMEM),
            pl.BlockSpec((1, 1, 1), lambda i: (i, 0, 0),
                         memory_space=pltpu.SMEM),
            pl.BlockSpec((1, 1, 1), lambda i: (i, 0, 0),
                         memory_space=pltpu.SMEM),
        ],
        out_shape=[
            jax.ShapeDtypeStruct((b, 1, d), jnp.float32),
            jax.ShapeDtypeStruct((b, 1, 1), jnp.int32),
            jax.ShapeDtypeStruct((b, 1, 1), jnp.float32),
            jax.ShapeDtypeStruct((b, 1, 1), jnp.float32),
        ],
    )(target_bboxes, tl, confT, locT, dbT)

    out_l, out_c = pl.pallas_call(
        _final_kernel,
        out_specs=[
            pl.BlockSpec(memory_space=pltpu.SMEM),
            pl.BlockSpec(memory_space=pltpu.SMEM),
        ],
        out_shape=[
            jax.ShapeDtypeStruct((1, 1), jnp.float32),
            jax.ShapeDtypeStruct((1, 1), jnp.float32),
        ],
    )(ce_neg, nump, posce, lossl)

    return (out_l[0, 0], out_c[0, 0])
